# baseline (speedup 1.0000x reference)
import functools

import jax
import jax.numpy as jnp
from jax import lax
from jax.experimental import pallas as pl
from jax.experimental.pallas import tpu as pltpu
from jax.experimental.pallas import tpu_sc as plsc

BATCH = 16384
EMB = 64
HID = 10
NC = 2
NS = 16
NW = NC * NS
BPW = BATCH // NW
CH = 256
NCH = BPW // CH
CG = CH // 16
CHUNK = 4
NKC = EMB // CHUNK
WPR = EMB // 2

P_B1 = 2 * EMB * HID
P_W2 = P_B1 + HID
P_B2 = P_W2 + HID
P_LEN = 1312

_mesh = plsc.VectorSubcoreMesh(core_axis_name="c", subcore_axis_name="s")


@functools.partial(
    pl.kernel,
    mesh=_mesh,
    out_type=jax.ShapeDtypeStruct((BATCH,), jnp.float32),
    compiler_params=pltpu.CompilerParams(needs_layout_passes=False),
    scratch_types=[
        pltpu.VMEM((BPW,), jnp.int32),
        pltpu.VMEM((BPW,), jnp.int32),
        pltpu.VMEM((BPW,), jnp.int32),
        pltpu.VMEM((BPW,), jnp.int32),
        pltpu.VMEM((128, 4 * WPR), jnp.int32),
        pltpu.VMEM((EMB, CH), jnp.float32),
        pltpu.VMEM((EMB, CH), jnp.float32),
        pltpu.VMEM((P_LEN,), jnp.float32),
        pltpu.VMEM((HID, BPW), jnp.float32),
        pltpu.VMEM((BPW,), jnp.float32),
        pltpu.SemaphoreType.DMA,
    ],
)
def _fwd(user_hbm, item_hbm, utab_hbm, itab_hbm, params_hbm, out_hbm,
         uq_v, iq_v, up_v, ip_v, bbuf_v, ubuf_v, ibuf_v, params_v,
         acc_v, out_v, sem):
    wid = lax.axis_index("s") * NC + lax.axis_index("c")
    base = wid * BPW

    pltpu.sync_copy(user_hbm.at[pl.ds(base, BPW)], uq_v)
    pltpu.sync_copy(item_hbm.at[pl.ds(base, BPW)], iq_v)
    pltpu.sync_copy(params_hbm, params_v)

    def split_body(g, carry):
        u = uq_v[pl.ds(g * 16, 16)]
        i = iq_v[pl.ds(g * 16, 16)]
        up_v[pl.ds(g * 16, 16)] = jnp.bitwise_and(u, 3)
        ip_v[pl.ds(g * 16, 16)] = jnp.bitwise_and(i, 3)
        uq_v[pl.ds(g * 16, 16)] = jnp.right_shift(u, 2)
        iq_v[pl.ds(g * 16, 16)] = jnp.right_shift(i, 2)
        return carry

    lax.fori_loop(0, BPW // 16, split_body, 0)

    def splat(j):
        return plsc.load_gather(
            params_v, [jnp.full((16,), 1, jnp.int32) * j])

    iota16 = lax.iota(jnp.int32, 16)

    binit = [splat(P_B1 + h) for h in range(HID)]

    def init_body(g, carry):
        for h in range(HID):
            acc_v[h, pl.ds(g * 16, 16)] = binit[h]
        return carry

    lax.fori_loop(0, BPW // 16, init_body, 0)

    def fetch_piece(tab_hbm, idx_v, par_v, dst_v, off):
        pltpu.async_copy(tab_hbm.at[idx_v.at[pl.ds(off, 128)]],
                         bbuf_v, sem).wait()
        drow = off % CH

        def grp_body(g, carry):
            pars = par_v[pl.ds(off + g * 16, 16)]
            for j in range(16):
                r = g * 16 + j
                w0 = pars[j] * WPR
                rcol = jnp.full((16,), 1, jnp.int32) * (drow + r)
                for c0 in (0, 1):
                    xi = bbuf_v[r, pl.ds(w0 + c0 * 16, 16)]
                    x32 = plsc.bitcast(xi, jnp.bfloat16)
                    a, b = plsc.unpack(
                        x32, format=plsc.PackFormat.INTERLEAVED)
                    fbase = c0 * 32
                    plsc.store_scatter(
                        dst_v, [fbase + 2 * iota16, rcol],
                        a.astype(jnp.float32))
                    plsc.store_scatter(
                        dst_v, [fbase + 1 + 2 * iota16, rcol],
                        b.astype(jnp.float32))
            return carry

        lax.fori_loop(0, 8, grp_body, 0)

    def make_l1(buf_ref, wbase, c):
        def l1_body(kc, carry):
            k0 = kc * CHUNK
            w = [[splat(wbase + (k0 + kk) * HID + h) for h in range(HID)]
                 for kk in range(CHUNK)]

            def g_body(g, carry):
                xs = []
                for kk in range(CHUNK):
                    xk = buf_ref[k0 + kk, pl.ds(g * 16, 16)]
                    xs.append(jnp.maximum(xk, 0.0))
                for h in range(HID):
                    a = acc_v[h, pl.ds(c * CH + g * 16, 16)]
                    for kk in range(CHUNK):
                        a = a + xs[kk] * w[kk][h]
                    acc_v[h, pl.ds(c * CH + g * 16, 16)] = a
                return carry

            lax.fori_loop(0, CG, g_body, 0)
            return carry

        return l1_body

    def chunk_loop(c, carry):
        for p in range(CH // 128):
            fetch_piece(utab_hbm, uq_v, up_v, ubuf_v, c * CH + p * 128)
        for p in range(CH // 128):
            fetch_piece(itab_hbm, iq_v, ip_v, ibuf_v, c * CH + p * 128)
        lax.fori_loop(0, NKC, make_l1(ubuf_v, 0, c), 0)
        lax.fori_loop(0, NKC, make_l1(ibuf_v, EMB * HID, c), 0)
        return carry

    lax.fori_loop(0, NCH, chunk_loop, 0)

    w2 = [splat(P_W2 + h) for h in range(HID)]
    b2v = splat(P_B2)

    def out_body(g, carry):
        o = b2v
        for h in range(HID):
            o = o + jnp.maximum(acc_v[h, pl.ds(g * 16, 16)], 0.0) * w2[h]
        out_v[pl.ds(g * 16, 16)] = o
        return carry

    lax.fori_loop(0, BPW // 16, out_body, 0)

    pltpu.sync_copy(out_v, out_hbm.at[pl.ds(base, BPW)])


def _pack_table(tab):
    n, d = tab.shape
    b = tab.astype(jnp.bfloat16).reshape(n // 4, 4 * (d // 2), 2)
    return jax.lax.bitcast_convert_type(b, jnp.int32)


def kernel(user, item, user_emb, item_emb, W1, b1, W2, b2):
    params = jnp.concatenate([
        W1.reshape(-1), b1.reshape(-1), W2.reshape(-1), b2.reshape(-1),
        jnp.zeros((P_LEN - P_B2 - 1,), jnp.float32),
    ])
    out = _fwd(user.astype(jnp.int32), item.astype(jnp.int32),
               _pack_table(user_emb), _pack_table(item_emb), params)
    return out.reshape(BATCH, 1)


# speedup vs baseline: 11.0974x; 11.0974x over previous
import functools

import jax
import jax.numpy as jnp
from jax import lax
from jax.experimental import pallas as pl
from jax.experimental.pallas import tpu as pltpu
from jax.experimental.pallas import tpu_sc as plsc

BATCH = 16384
EMB = 64
HID = 10
NC = 2
NS = 16
NW = NC * NS
BPW = BATCH // NW
CH = 256
NCH = BPW // CH
CG = CH // 16
CHUNK = 4
NKC = EMB // CHUNK
WPR = EMB // 2

P_B1 = 2 * EMB * HID
P_W2 = P_B1 + HID
P_B2 = P_W2 + HID
P_LEN = 1312

_mesh = plsc.VectorSubcoreMesh(core_axis_name="c", subcore_axis_name="s")


@functools.partial(
    pl.kernel,
    mesh=_mesh,
    out_type=jax.ShapeDtypeStruct((BATCH,), jnp.float32),
    compiler_params=pltpu.CompilerParams(needs_layout_passes=False),
    scratch_types=[
        pltpu.VMEM((BPW,), jnp.int32),
        pltpu.VMEM((BPW,), jnp.int32),
        pltpu.VMEM((BPW,), jnp.int32),
        pltpu.VMEM((BPW,), jnp.int32),
        pltpu.VMEM((128, 4 * WPR), jnp.int32),
        pltpu.VMEM((EMB, CH), jnp.float32),
        pltpu.VMEM((EMB, CH), jnp.float32),
        pltpu.VMEM((P_LEN,), jnp.float32),
        pltpu.VMEM((HID, BPW), jnp.float32),
        pltpu.VMEM((BPW,), jnp.float32),
        pltpu.SemaphoreType.DMA,
    ],
)
def _fwd(user_hbm, item_hbm, utab_hbm, itab_hbm, params_hbm, out_hbm,
         uq_v, iq_v, up_v, ip_v, bbuf_v, ubuf_v, ibuf_v, params_v,
         acc_v, out_v, sem):
    wid = lax.axis_index("s") * NC + lax.axis_index("c")
    base = wid * BPW

    pltpu.sync_copy(user_hbm.at[pl.ds(base, BPW)], uq_v)
    pltpu.sync_copy(item_hbm.at[pl.ds(base, BPW)], iq_v)
    pltpu.sync_copy(params_hbm, params_v)

    def split_body(g, carry):
        u = uq_v[pl.ds(g * 16, 16)]
        i = iq_v[pl.ds(g * 16, 16)]
        up_v[pl.ds(g * 16, 16)] = jnp.bitwise_and(u, 3)
        ip_v[pl.ds(g * 16, 16)] = jnp.bitwise_and(i, 3)
        uq_v[pl.ds(g * 16, 16)] = jnp.right_shift(u, 2)
        iq_v[pl.ds(g * 16, 16)] = jnp.right_shift(i, 2)
        return carry

    lax.fori_loop(0, BPW // 16, split_body, 0)

    def splat(j):
        return plsc.load_gather(
            params_v, [jnp.full((16,), 1, jnp.int32) * j])

    iota16 = lax.iota(jnp.int32, 16)

    binit = [splat(P_B1 + h) for h in range(HID)]

    def init_body(g, carry):
        for h in range(HID):
            acc_v[h, pl.ds(g * 16, 16)] = binit[h]
        return carry

    lax.fori_loop(0, BPW // 16, init_body, 0)

    def fetch_piece(tab_hbm, idx_v, par_v, dst_v, off):
        pltpu.async_copy(tab_hbm.at[idx_v.at[pl.ds(off, 128)]],
                         bbuf_v, sem).wait()
        drow = off % CH

        def grp_body(g, carry):
            pars = par_v[pl.ds(off + g * 16, 16)]
            for j in range(16):
                r = g * 16 + j
                w0 = pars[j] * WPR
                rcol = jnp.full((16,), 1, jnp.int32) * (drow + r)
                for c0 in (0, 1):
                    xi = bbuf_v[r, pl.ds(w0 + c0 * 16, 16)]
                    x32 = plsc.bitcast(xi, jnp.bfloat16)
                    a, b = plsc.unpack(
                        x32, format=plsc.PackFormat.INTERLEAVED)
                    fbase = c0 * 32
                    plsc.store_scatter(
                        dst_v, [fbase + 2 * iota16, rcol],
                        a.astype(jnp.float32))
                    plsc.store_scatter(
                        dst_v, [fbase + 1 + 2 * iota16, rcol],
                        b.astype(jnp.float32))
            return carry

        lax.fori_loop(0, 8, grp_body, 0)

    def make_l1(buf_ref, wbase, c):
        def l1_body(kc, carry):
            k0 = kc * CHUNK
            w = [[splat(wbase + (k0 + kk) * HID + h) for h in range(HID)]
                 for kk in range(CHUNK)]

            def g_body(g, carry):
                xs = []
                for kk in range(CHUNK):
                    xk = buf_ref[k0 + kk, pl.ds(g * 16, 16)]
                    xs.append(jnp.maximum(xk, 0.0))
                for h in range(HID):
                    a = acc_v[h, pl.ds(c * CH + g * 16, 16)]
                    for kk in range(CHUNK):
                        a = a + xs[kk] * w[kk][h]
                    acc_v[h, pl.ds(c * CH + g * 16, 16)] = a
                return carry

            lax.fori_loop(0, CG, g_body, 0)
            return carry

        return l1_body

    def chunk_loop(c, carry):
        for p in range(CH // 128):
            fetch_piece(utab_hbm, uq_v, up_v, ubuf_v, c * CH + p * 128)
        for p in range(CH // 128):
            fetch_piece(itab_hbm, iq_v, ip_v, ibuf_v, c * CH + p * 128)
        lax.fori_loop(0, NKC, make_l1(ubuf_v, 0, c), 0)
        lax.fori_loop(0, NKC, make_l1(ibuf_v, EMB * HID, c), 0)
        return carry

    lax.fori_loop(0, NCH, chunk_loop, 0)

    w2 = [splat(P_W2 + h) for h in range(HID)]
    b2v = splat(P_B2)

    def out_body(g, carry):
        o = b2v
        for h in range(HID):
            o = o + jnp.maximum(acc_v[h, pl.ds(g * 16, 16)], 0.0) * w2[h]
        out_v[pl.ds(g * 16, 16)] = o
        return carry

    lax.fori_loop(0, BPW // 16, out_body, 0)

    pltpu.sync_copy(out_v, out_hbm.at[pl.ds(base, BPW)])


def _pack_table(tab):
    n, d = tab.shape
    u16 = jax.lax.bitcast_convert_type(tab.astype(jnp.bfloat16), jnp.uint16)
    lo = u16[:, 0::2].astype(jnp.uint32)
    hi = u16[:, 1::2].astype(jnp.uint32)
    w = lo | (hi << 16)
    w3 = w.reshape(n // 4, 4, d // 2)
    pt = jnp.transpose(w3, (1, 2, 0)).reshape(2 * d, n // 4)
    packed_t = jax.lax.optimization_barrier(pt)
    return jnp.transpose(packed_t).astype(jnp.int32)


def kernel(user, item, user_emb, item_emb, W1, b1, W2, b2):
    params = jnp.concatenate([
        W1.reshape(-1), b1.reshape(-1), W2.reshape(-1), b2.reshape(-1),
        jnp.zeros((P_LEN - P_B2 - 1,), jnp.float32),
    ])
    out = _fwd(user.astype(jnp.int32), item.astype(jnp.int32),
               _pack_table(user_emb), _pack_table(item_emb), params)
    return out.reshape(BATCH, 1)


# speedup vs baseline: 53.9028x; 4.8573x over previous
import functools

import jax
import jax.numpy as jnp
from jax import lax
from jax.experimental import pallas as pl
from jax.experimental.pallas import tpu as pltpu
from jax.experimental.pallas import tpu_sc as plsc

BATCH = 16384
EMB = 64
HID = 10
NC = 2
NS = 16
NW = NC * NS
BPW = BATCH // NW
CH = 128
NCH = BPW // CH
CG = CH // 16
CHUNK = 4
NKC = EMB // CHUNK

P_B1 = 2 * EMB * HID
P_W2 = P_B1 + HID
P_B2 = P_W2 + HID
P_LEN = 1312

_mesh = plsc.VectorSubcoreMesh(core_axis_name="c", subcore_axis_name="s")


@functools.partial(
    pl.kernel,
    mesh=_mesh,
    out_type=jax.ShapeDtypeStruct((BATCH,), jnp.float32),
    compiler_params=pltpu.CompilerParams(needs_layout_passes=False),
    scratch_types=[
        pltpu.VMEM((BPW,), jnp.int32),
        pltpu.VMEM((BPW,), jnp.int32),
        pltpu.VMEM((2, CH, EMB), jnp.float32),
        pltpu.VMEM((2, CH, EMB), jnp.float32),
        pltpu.VMEM((P_LEN,), jnp.float32),
        pltpu.VMEM((HID, BPW), jnp.float32),
        pltpu.VMEM((BPW,), jnp.float32),
        pltpu.SemaphoreType.DMA,
        pltpu.SemaphoreType.DMA,
    ],
)
def _fwd(user_hbm, item_hbm, uemb_hbm, iemb_hbm, params_hbm, out_hbm,
         uidx_v, iidx_v, urows_v, irows_v, params_v, acc_v, out_v,
         sem_u, sem_i):
    wid = lax.axis_index("s") * NC + lax.axis_index("c")
    base = wid * BPW

    pltpu.sync_copy(user_hbm.at[pl.ds(base, BPW)], uidx_v)
    pltpu.sync_copy(item_hbm.at[pl.ds(base, BPW)], iidx_v)
    pltpu.sync_copy(params_hbm, params_v)

    def splat(j):
        return plsc.load_gather(
            params_v, [jnp.full((16,), 1, jnp.int32) * j])

    iota16 = lax.iota(jnp.int32, 16)

    binit = [splat(P_B1 + h) for h in range(HID)]

    def init_body(g, carry):
        for h in range(HID):
            acc_v[h, pl.ds(g * 16, 16)] = binit[h]
        return carry

    lax.fori_loop(0, BPW // 16, init_body, 0)

    def issue_chunk(c, buf):
        def issue_body(g, carry):
            uiv = uidx_v[pl.ds(c * CH + g * 16, 16)]
            iiv = iidx_v[pl.ds(c * CH + g * 16, 16)]
            for j in range(16):
                r = g * 16 + j
                pltpu.async_copy(
                    uemb_hbm.at[uiv[j]], urows_v.at[buf, r], sem_u)
                pltpu.async_copy(
                    iemb_hbm.at[iiv[j]], irows_v.at[buf, r], sem_i)
            return carry

        lax.fori_loop(0, CG, issue_body, 0)

    def drain_chunk():
        def drain_body(r, carry):
            pltpu.make_async_copy(
                uemb_hbm.at[0], urows_v.at[0, 0], sem_u).wait()
            pltpu.make_async_copy(
                iemb_hbm.at[0], irows_v.at[0, 0], sem_i).wait()
            return carry

        lax.fori_loop(0, CH, drain_body, 0)

    def make_l1(rows_ref, wbase, c, buf):
        def l1_body(kc, carry):
            k0 = kc * CHUNK
            w = [[splat(wbase + (k0 + kk) * HID + h) for h in range(HID)]
                 for kk in range(CHUNK)]

            def g_body(g, carry):
                rows = g * 16 + iota16
                xs = []
                for kk in range(CHUNK):
                    col = jnp.full((16,), kk, jnp.int32) + k0
                    xk = plsc.load_gather(rows_ref.at[buf], [rows, col])
                    xs.append(jnp.maximum(xk, 0.0))
                for h in range(HID):
                    a = acc_v[h, pl.ds(c * CH + g * 16, 16)]
                    for kk in range(CHUNK):
                        a = a + xs[kk] * w[kk][h]
                    acc_v[h, pl.ds(c * CH + g * 16, 16)] = a
                return carry

            lax.fori_loop(0, CG, g_body, 0)
            return carry

        return l1_body

    issue_chunk(0, 0)

    def chunk_loop(c, carry):
        buf = lax.rem(c, 2)
        drain_chunk()

        @pl.when(c < NCH - 1)
        def _():
            issue_chunk(c + 1, 1 - buf)

        lax.fori_loop(0, NKC, make_l1(urows_v, 0, c, buf), 0)
        lax.fori_loop(0, NKC, make_l1(irows_v, EMB * HID, c, buf), 0)
        return carry

    lax.fori_loop(0, NCH, chunk_loop, 0)

    w2 = [splat(P_W2 + h) for h in range(HID)]
    b2v = splat(P_B2)

    def out_body(g, carry):
        o = b2v
        for h in range(HID):
            o = o + jnp.maximum(acc_v[h, pl.ds(g * 16, 16)], 0.0) * w2[h]
        out_v[pl.ds(g * 16, 16)] = o
        return carry

    lax.fori_loop(0, BPW // 16, out_body, 0)

    pltpu.sync_copy(out_v, out_hbm.at[pl.ds(base, BPW)])


def kernel(user, item, user_emb, item_emb, W1, b1, W2, b2):
    params = jnp.concatenate([
        W1.reshape(-1), b1.reshape(-1), W2.reshape(-1), b2.reshape(-1),
        jnp.zeros((P_LEN - P_B2 - 1,), jnp.float32),
    ])
    out = _fwd(user.astype(jnp.int32), item.astype(jnp.int32),
               user_emb, item_emb, params)
    return out.reshape(BATCH, 1)
